# baseline (speedup 1.0000x reference)
import jax
import jax.numpy as jnp
from jax.experimental import pallas as pl
from jax.experimental.pallas import tpu as pltpu


def _fused_body(x_ref, w_ref, o_ref, *, inv_hw):
    xb = x_ref[0]
    mean = jnp.sum(xb, axis=1, keepdims=True) * inv_hw
    y = jax.lax.dot_general(
        w_ref[...], mean,
        dimension_numbers=(((1,), (0,)), ((), ())),
        preferred_element_type=jnp.float32,
    )
    y = jnp.maximum(y, 0.0)
    o_ref[0] = jnp.broadcast_to(y, o_ref.shape[1:]).astype(o_ref.dtype)


def kernel(x, weight):
    n, c_in, h, w = x.shape
    c_out = weight.shape[0]
    hw = h * w
    x_flat = x.reshape(n, c_in, hw)
    w2d = weight.reshape(c_out, c_in)

    import functools
    body = functools.partial(_fused_body, inv_hw=float(1.0 / hw))

    out_flat = pl.pallas_call(
        body,
        out_shape=jax.ShapeDtypeStruct((n, c_out, hw), x.dtype),
        grid=(n,),
        in_specs=[
            pl.BlockSpec((1, c_in, hw), lambda i: (i, 0, 0)),
            pl.BlockSpec((c_out, c_in), lambda i: (0, 0)),
        ],
        out_specs=pl.BlockSpec((1, c_out, hw), lambda i: (i, 0, 0)),
        compiler_params=pltpu.CompilerParams(
            dimension_semantics=("parallel",),
            vmem_limit_bytes=56 * 1024 * 1024,
        ),
    )(x_flat, w2d)
    return out_flat.reshape(n, c_out, h, w)
